# Initial kernel scaffold; baseline (speedup 1.0000x reference)
import jax
import jax.numpy as jnp
from jax.experimental import pallas as pl


def kernel(x, conv1_1_w, bn1_1_g, bn1_1_be, bn1_1_m, bn1_1_v, conv1_2_w, conv2_1_w, bn2_1_g, bn2_1_be, bn2_1_m, bn2_1_v, conv2_2_w, bn2_2_g, bn2_2_be, bn2_2_m, bn2_2_v, conv2_3_w, bn_concat_g, bn_concat_be, bn_concat_m, bn_concat_v, conv_w, bn_g, bn_be, bn_m, bn_v, fc1_w, fc1_b, fc2_w, fc2_b):
    raise NotImplementedError("write your pallas kernel here")


# speedup vs baseline: 1.0838x; 1.0838x over previous
import functools

import jax
import jax.numpy as jnp
from jax.experimental import pallas as pl
from jax.experimental.pallas import tpu as pltpu


def _fold_bn(gamma, beta, mean, var, eps=1e-5):
    scale = gamma / jnp.sqrt(var + eps)
    return scale, beta - mean * scale


def _dense_grouped(wg):
    G, kh, kw, ci, co = wg.shape
    w = jnp.zeros((kh * kw, G * ci, G * co), wg.dtype)
    for g in range(G):
        w = w.at[:, g * ci:(g + 1) * ci, g * co:(g + 1) * co].set(
            wg[g].reshape(kh * kw, ci, co))
    return w


def _rep_lanes(m, cout):
    k = cout // m.shape[1]
    if k == 1:
        return m
    return jnp.concatenate([m] * k, axis=1)


def _block_kernel(H, W, B, DC2, DC,
                  x_ref, ws_ref, bs_ref,
                  w1_ref, s1_ref, w2_ref, s2_ref, w3_ref, s3_ref,
                  wfa_ref, wfb_ref, sf_ref,
                  f1w_ref, f1b_ref, f2w_ref, f2b_ref,
                  o_ref):
    f32 = jnp.float32
    bf16 = jnp.bfloat16
    HW = H * W
    R = B * HW
    Cout = o_ref.shape[1]

    x = x_ref[...]
    stem = jnp.dot(x.astype(bf16), ws_ref[...], preferred_element_type=f32)
    stem = jnp.maximum(stem + bs_ref[...], 0.0)

    MW = min(128, DC)
    p = jax.lax.broadcasted_iota(jnp.int32, (R, MW), 0)
    w_pos = jax.lax.rem(p, W)
    h_pos = jax.lax.rem(jax.lax.div(p, W), H)
    one, zero = jnp.float32(1.0), jnp.float32(0.0)
    m_h = [jnp.where(h_pos <= H - 2, one, zero), None,
           jnp.where(h_pos >= 1, one, zero)]
    m_w = [jnp.where(w_pos <= W - 2, one, zero), None,
           jnp.where(w_pos >= 1, one, zero)]

    def conv3x3(src, w9_ref, sh_ref, cout):
        srcb = src.astype(bf16)
        acc = None
        for dy in range(3):
            for dx in range(3):
                off = (dy - 1) * W + (dx - 1)
                y = jnp.dot(srcb, w9_ref[dy * 3 + dx],
                            preferred_element_type=f32)
                if m_h[dy] is not None:
                    y = y * _rep_lanes(m_h[dy], cout)
                if m_w[dx] is not None:
                    y = y * _rep_lanes(m_w[dx], cout)
                lo = max(0, -off)
                hi = R - max(0, off)
                z = y[lo + off:hi + off, :]
                if lo or hi != R:
                    parts = []
                    if lo:
                        parts.append(jnp.zeros((lo, cout), f32))
                    parts.append(z)
                    if hi != R:
                        parts.append(jnp.zeros((R - hi, cout), f32))
                    z = jnp.concatenate(parts, axis=0)
                acc = z if acc is None else acc + z
        return jnp.maximum(acc + sh_ref[...], 0.0)

    t1 = stem[:, :DC2]
    t2 = stem[:, DC2:]
    b1 = conv3x3(t1, w1_ref, s1_ref, DC2)
    b2 = conv3x3(t2, w2_ref, s2_ref, DC)
    b2 = conv3x3(b2, w3_ref, s3_ref, DC)

    out = jnp.dot(b1.astype(bf16), wfa_ref[...], preferred_element_type=f32)
    out = out + jnp.dot(b2.astype(bf16), wfb_ref[...], preferred_element_type=f32)
    out = out + sf_ref[...]

    img_of_row = jax.lax.div(jax.lax.broadcasted_iota(jnp.int32, (B, R), 1), HW)
    sel_pool = jnp.where(img_of_row == jax.lax.broadcasted_iota(jnp.int32, (B, R), 0),
                         one, zero)
    pooled = jnp.dot(sel_pool, out, preferred_element_type=f32) * (1.0 / HW)
    se = jnp.maximum(jnp.dot(pooled, f1w_ref[...], preferred_element_type=f32)
                     + f1b_ref[...], 0.0)
    se = jax.nn.sigmoid(jnp.dot(se, f2w_ref[...], preferred_element_type=f32)
                        + f2b_ref[...])

    row_img = jax.lax.div(jax.lax.broadcasted_iota(jnp.int32, (R, B), 0), HW)
    sel_bcast = jnp.where(row_img == jax.lax.broadcasted_iota(jnp.int32, (R, B), 1),
                          one, zero)
    se_rows = jnp.dot(sel_bcast, se, preferred_element_type=f32)

    o_ref[...] = jnp.maximum(out * se_rows + x, 0.0)


def kernel(x, conv1_1_w, bn1_1_g, bn1_1_be, bn1_1_m, bn1_1_v, conv1_2_w,
           conv2_1_w, bn2_1_g, bn2_1_be, bn2_1_m, bn2_1_v, conv2_2_w,
           bn2_2_g, bn2_2_be, bn2_2_m, bn2_2_v, conv2_3_w,
           bn_concat_g, bn_concat_be, bn_concat_m, bn_concat_v,
           conv_w, bn_g, bn_be, bn_m, bn_v, fc1_w, fc1_b, fc2_w, fc2_b):
    f32, bf16 = jnp.float32, jnp.bfloat16
    N, Cin, H, W = x.shape
    HW = H * W
    DC2 = conv1_1_w.shape[1]
    DC = conv2_1_w.shape[1]
    DC3 = DC2 + DC
    Cout = conv_w.shape[1]
    Cr = fc1_w.shape[1]

    s11, h11 = _fold_bn(bn1_1_g, bn1_1_be, bn1_1_m, bn1_1_v)
    s21, h21 = _fold_bn(bn2_1_g, bn2_1_be, bn2_1_m, bn2_1_v)
    s22, h22 = _fold_bn(bn2_2_g, bn2_2_be, bn2_2_m, bn2_2_v)
    scat, hcat = _fold_bn(bn_concat_g, bn_concat_be, bn_concat_m, bn_concat_v)
    sbn, hbn = _fold_bn(bn_g, bn_be, bn_m, bn_v)

    ws = jnp.concatenate([conv1_1_w * s11, conv2_1_w * s21], axis=1).astype(bf16)
    bs = jnp.concatenate([h11, h21]).reshape(1, DC3)
    w1 = (_dense_grouped(conv1_2_w) * scat[:DC2]).astype(bf16)
    w2 = (_dense_grouped(conv2_2_w) * s22).astype(bf16)
    w3 = (_dense_grouped(conv2_3_w) * scat[DC2:]).astype(bf16)
    wfa = (conv_w[:DC2] * sbn).astype(bf16)
    wfb = (conv_w[DC2:] * sbn).astype(bf16)

    x2 = jnp.transpose(x, (0, 2, 3, 1)).astype(f32).reshape(N * HW, Cin)
    B = next(b for b in (8, 6, 4, 3, 2, 1) if N % b == 0)
    R = B * HW
    G = N // B

    kfn = functools.partial(_block_kernel, H, W, B, DC2, DC)
    full = lambda i: (0, 0)
    full3 = lambda i: (0, 0, 0)
    flops = N * (2 * HW * Cin * DC3
                 + 2 * HW * 9 * (DC2 * DC2 + 2 * DC * DC)
                 + 2 * HW * DC3 * Cout
                 + 4 * Cout * Cr + 5 * HW * Cout)
    bytes_acc = (N * HW * (Cin + Cout) * 4
                 + 2 * (Cin * DC3 + 9 * (DC2 * DC2 + 2 * DC * DC) + DC3 * Cout)
                 + 4 * (DC3 + DC2 + 2 * DC + 2 * Cout + 2 * Cout * Cr + Cr))

    out = pl.pallas_call(
        kfn,
        out_shape=jax.ShapeDtypeStruct((N * HW, Cout), f32),
        grid=(G,),
        in_specs=[
            pl.BlockSpec((R, Cin), lambda i: (i, 0)),
            pl.BlockSpec((Cin, DC3), full),
            pl.BlockSpec((1, DC3), full),
            pl.BlockSpec((9, DC2, DC2), full3),
            pl.BlockSpec((1, DC2), full),
            pl.BlockSpec((9, DC, DC), full3),
            pl.BlockSpec((1, DC), full),
            pl.BlockSpec((9, DC, DC), full3),
            pl.BlockSpec((1, DC), full),
            pl.BlockSpec((DC2, Cout), full),
            pl.BlockSpec((DC, Cout), full),
            pl.BlockSpec((1, Cout), full),
            pl.BlockSpec((Cout, Cr), full),
            pl.BlockSpec((1, Cr), full),
            pl.BlockSpec((Cr, Cout), full),
            pl.BlockSpec((1, Cout), full),
        ],
        out_specs=pl.BlockSpec((R, Cout), lambda i: (i, 0)),
        compiler_params=pltpu.CompilerParams(
            dimension_semantics=("parallel",),
            vmem_limit_bytes=64 * 1024 * 1024),
        cost_estimate=pl.CostEstimate(flops=flops, transcendentals=N * Cout,
                                      bytes_accessed=bytes_acc),
    )(x2, ws, bs,
      w1, hcat[:DC2].reshape(1, DC2),
      w2, h22.reshape(1, DC),
      w3, hcat[DC2:].reshape(1, DC),
      wfa, wfb, hbn.reshape(1, Cout),
      fc1_w, fc1_b.reshape(1, Cr),
      fc2_w, fc2_b.reshape(1, Cout))

    return jnp.transpose(out.reshape(N, H, W, Cout), (0, 3, 1, 2))


# speedup vs baseline: 2.1081x; 1.9450x over previous
import functools

import jax
import jax.numpy as jnp
from jax.experimental import pallas as pl
from jax.experimental.pallas import tpu as pltpu


def _fold_bn(gamma, beta, mean, var, eps=1e-5):
    scale = gamma / jnp.sqrt(var + eps)
    return scale, beta - mean * scale


def _dense_grouped(wg):
    G, kh, kw, ci, co = wg.shape
    w9 = jnp.transpose(wg, (1, 2, 0, 3, 4)).reshape(kh * kw, G, ci, co)
    eye = jnp.eye(G, dtype=wg.dtype)
    dense = w9[:, :, :, None, :] * eye[None, :, None, :, None]
    return dense.reshape(kh * kw, G * ci, G * co)


def _rep_lanes(m, cout):
    k = cout // m.shape[1]
    if k == 1:
        return m
    return jnp.concatenate([m] * k, axis=1)


def _block_kernel(H, W, B, DC2, DC,
                  x_ref, ws_ref, bs_ref,
                  w1_ref, s1_ref, w2_ref, s2_ref, w3_ref, s3_ref,
                  wfa_ref, wfb_ref, sf_ref,
                  f1w_ref, f1b_ref, f2w_ref, f2b_ref,
                  o_ref):
    f32 = jnp.float32
    bf16 = jnp.bfloat16
    HW = H * W
    R = B * HW
    Cout = o_ref.shape[1]

    x = x_ref[...]
    stem = jnp.dot(x.astype(bf16), ws_ref[...], preferred_element_type=f32)
    stem = jnp.maximum(stem + bs_ref[...], 0.0)

    MW = min(128, DC)
    p = jax.lax.broadcasted_iota(jnp.int32, (R, MW), 0)
    w_pos = jax.lax.rem(p, W)
    h_pos = jax.lax.rem(jax.lax.div(p, W), H)
    one, zero = jnp.float32(1.0), jnp.float32(0.0)
    m_h = [jnp.where(h_pos <= H - 2, one, zero), None,
           jnp.where(h_pos >= 1, one, zero)]
    m_w = [jnp.where(w_pos <= W - 2, one, zero), None,
           jnp.where(w_pos >= 1, one, zero)]

    def conv3x3(src, w9_ref, sh_ref, cout):
        srcb = src.astype(bf16)
        acc = None
        for dy in range(3):
            for dx in range(3):
                off = (dy - 1) * W + (dx - 1)
                y = jnp.dot(srcb, w9_ref[dy * 3 + dx],
                            preferred_element_type=f32)
                if m_h[dy] is not None:
                    y = y * _rep_lanes(m_h[dy], cout)
                if m_w[dx] is not None:
                    y = y * _rep_lanes(m_w[dx], cout)
                lo = max(0, -off)
                hi = R - max(0, off)
                z = y[lo + off:hi + off, :]
                if lo or hi != R:
                    parts = []
                    if lo:
                        parts.append(jnp.zeros((lo, cout), f32))
                    parts.append(z)
                    if hi != R:
                        parts.append(jnp.zeros((R - hi, cout), f32))
                    z = jnp.concatenate(parts, axis=0)
                acc = z if acc is None else acc + z
        return jnp.maximum(acc + sh_ref[...], 0.0)

    t1 = stem[:, :DC2]
    t2 = stem[:, DC2:]
    b1 = conv3x3(t1, w1_ref, s1_ref, DC2)
    b2 = conv3x3(t2, w2_ref, s2_ref, DC)
    b2 = conv3x3(b2, w3_ref, s3_ref, DC)

    out = jnp.dot(b1.astype(bf16), wfa_ref[...], preferred_element_type=f32)
    out = out + jnp.dot(b2.astype(bf16), wfb_ref[...], preferred_element_type=f32)
    out = out + sf_ref[...]

    img_of_row = jax.lax.div(jax.lax.broadcasted_iota(jnp.int32, (B, R), 1), HW)
    sel_pool = jnp.where(img_of_row == jax.lax.broadcasted_iota(jnp.int32, (B, R), 0),
                         one, zero)
    pooled = jnp.dot(sel_pool, out, preferred_element_type=f32) * (1.0 / HW)
    se = jnp.maximum(jnp.dot(pooled, f1w_ref[...], preferred_element_type=f32)
                     + f1b_ref[...], 0.0)
    se = jax.nn.sigmoid(jnp.dot(se, f2w_ref[...], preferred_element_type=f32)
                        + f2b_ref[...])

    row_img = jax.lax.div(jax.lax.broadcasted_iota(jnp.int32, (R, B), 0), HW)
    sel_bcast = jnp.where(row_img == jax.lax.broadcasted_iota(jnp.int32, (R, B), 1),
                          one, zero)
    se_rows = jnp.dot(sel_bcast, se, preferred_element_type=f32)

    o_ref[...] = jnp.maximum(out * se_rows + x, 0.0)


def kernel(x, conv1_1_w, bn1_1_g, bn1_1_be, bn1_1_m, bn1_1_v, conv1_2_w,
           conv2_1_w, bn2_1_g, bn2_1_be, bn2_1_m, bn2_1_v, conv2_2_w,
           bn2_2_g, bn2_2_be, bn2_2_m, bn2_2_v, conv2_3_w,
           bn_concat_g, bn_concat_be, bn_concat_m, bn_concat_v,
           conv_w, bn_g, bn_be, bn_m, bn_v, fc1_w, fc1_b, fc2_w, fc2_b):
    f32, bf16 = jnp.float32, jnp.bfloat16
    N, Cin, H, W = x.shape
    HW = H * W
    DC2 = conv1_1_w.shape[1]
    DC = conv2_1_w.shape[1]
    DC3 = DC2 + DC
    Cout = conv_w.shape[1]
    Cr = fc1_w.shape[1]

    s11, h11 = _fold_bn(bn1_1_g, bn1_1_be, bn1_1_m, bn1_1_v)
    s21, h21 = _fold_bn(bn2_1_g, bn2_1_be, bn2_1_m, bn2_1_v)
    s22, h22 = _fold_bn(bn2_2_g, bn2_2_be, bn2_2_m, bn2_2_v)
    scat, hcat = _fold_bn(bn_concat_g, bn_concat_be, bn_concat_m, bn_concat_v)
    sbn, hbn = _fold_bn(bn_g, bn_be, bn_m, bn_v)

    ws = jnp.concatenate([conv1_1_w * s11, conv2_1_w * s21], axis=1).astype(bf16)
    bs = jnp.concatenate([h11, h21]).reshape(1, DC3)
    w1 = (_dense_grouped(conv1_2_w) * scat[:DC2]).astype(bf16)
    w2 = (_dense_grouped(conv2_2_w) * s22).astype(bf16)
    w3 = (_dense_grouped(conv2_3_w) * scat[DC2:]).astype(bf16)
    wfa = (conv_w[:DC2] * sbn).astype(bf16)
    wfb = (conv_w[DC2:] * sbn).astype(bf16)

    x2 = jnp.transpose(x, (0, 2, 3, 1)).astype(f32).reshape(N * HW, Cin)
    B = next(b for b in (8, 6, 4, 3, 2, 1) if N % b == 0)
    R = B * HW
    G = N // B

    kfn = functools.partial(_block_kernel, H, W, B, DC2, DC)
    full = lambda i: (0, 0)
    full3 = lambda i: (0, 0, 0)
    flops = N * (2 * HW * Cin * DC3
                 + 2 * HW * 9 * (DC2 * DC2 + 2 * DC * DC)
                 + 2 * HW * DC3 * Cout
                 + 4 * Cout * Cr + 5 * HW * Cout)
    bytes_acc = (N * HW * (Cin + Cout) * 4
                 + 2 * (Cin * DC3 + 9 * (DC2 * DC2 + 2 * DC * DC) + DC3 * Cout)
                 + 4 * (DC3 + DC2 + 2 * DC + 2 * Cout + 2 * Cout * Cr + Cr))

    out = pl.pallas_call(
        kfn,
        out_shape=jax.ShapeDtypeStruct((N * HW, Cout), f32),
        grid=(G,),
        in_specs=[
            pl.BlockSpec((R, Cin), lambda i: (i, 0)),
            pl.BlockSpec((Cin, DC3), full),
            pl.BlockSpec((1, DC3), full),
            pl.BlockSpec((9, DC2, DC2), full3),
            pl.BlockSpec((1, DC2), full),
            pl.BlockSpec((9, DC, DC), full3),
            pl.BlockSpec((1, DC), full),
            pl.BlockSpec((9, DC, DC), full3),
            pl.BlockSpec((1, DC), full),
            pl.BlockSpec((DC2, Cout), full),
            pl.BlockSpec((DC, Cout), full),
            pl.BlockSpec((1, Cout), full),
            pl.BlockSpec((Cout, Cr), full),
            pl.BlockSpec((1, Cr), full),
            pl.BlockSpec((Cr, Cout), full),
            pl.BlockSpec((1, Cout), full),
        ],
        out_specs=pl.BlockSpec((R, Cout), lambda i: (i, 0)),
        compiler_params=pltpu.CompilerParams(
            dimension_semantics=("parallel",),
            vmem_limit_bytes=64 * 1024 * 1024),
        cost_estimate=pl.CostEstimate(flops=flops, transcendentals=N * Cout,
                                      bytes_accessed=bytes_acc),
    )(x2, ws, bs,
      w1, hcat[:DC2].reshape(1, DC2),
      w2, h22.reshape(1, DC),
      w3, hcat[DC2:].reshape(1, DC),
      wfa, wfb, hbn.reshape(1, Cout),
      fc1_w, fc1_b.reshape(1, Cr),
      fc2_w, fc2_b.reshape(1, Cout))

    return jnp.transpose(out.reshape(N, H, W, Cout), (0, 3, 1, 2))


# speedup vs baseline: 2.1096x; 1.0007x over previous
import functools

import jax
import jax.numpy as jnp
from jax.experimental import pallas as pl
from jax.experimental.pallas import tpu as pltpu


def _fold_bn(gamma, beta, mean, var, eps=1e-5):
    scale = gamma / jnp.sqrt(var + eps)
    return scale, beta - mean * scale


def _dense_grouped(wg):
    G, kh, kw, ci, co = wg.shape
    w9 = jnp.transpose(wg, (1, 2, 0, 3, 4)).reshape(kh * kw, G, ci, co)
    eye = jnp.eye(G, dtype=wg.dtype)
    dense = w9[:, :, :, None, :] * eye[None, :, None, :, None]
    return dense.reshape(kh * kw, G * ci, G * co)


def _rep_lanes(m, cout):
    k = cout // m.shape[1]
    if k == 1:
        return m
    return jnp.concatenate([m] * k, axis=1)


def _block_kernel(H, W, B, DC2, DC,
                  x_ref, ws_ref, bs_ref,
                  w1_ref, s1_ref, w2_ref, s2_ref, w3_ref, s3_ref,
                  wfa_ref, wfb_ref, sf_ref,
                  f1w_ref, f1b_ref, f2w_ref, f2b_ref,
                  o_ref):
    f32 = jnp.float32
    bf16 = jnp.bfloat16
    HW = H * W
    R = B * HW
    Cout = o_ref.shape[2]

    x = jnp.concatenate([jnp.transpose(x_ref[b], (1, 0)) for b in range(B)],
                        axis=0)
    stem = jnp.dot(x.astype(bf16), ws_ref[...], preferred_element_type=f32)
    stem = jnp.maximum(stem + bs_ref[...], 0.0)

    MW = min(128, DC)
    p = jax.lax.broadcasted_iota(jnp.int32, (R, MW), 0)
    w_pos = jax.lax.rem(p, W)
    h_pos = jax.lax.rem(jax.lax.div(p, W), H)
    one, zero = jnp.float32(1.0), jnp.float32(0.0)
    m_h = [jnp.where(h_pos <= H - 2, one, zero), None,
           jnp.where(h_pos >= 1, one, zero)]
    m_w = [jnp.where(w_pos <= W - 2, one, zero), None,
           jnp.where(w_pos >= 1, one, zero)]

    def conv3x3(src, w9_ref, sh_ref, cout):
        srcb = src.astype(bf16)
        acc = None
        for dy in range(3):
            for dx in range(3):
                off = (dy - 1) * W + (dx - 1)
                y = jnp.dot(srcb, w9_ref[dy * 3 + dx],
                            preferred_element_type=f32)
                if m_h[dy] is not None:
                    y = y * _rep_lanes(m_h[dy], cout)
                if m_w[dx] is not None:
                    y = y * _rep_lanes(m_w[dx], cout)
                lo = max(0, -off)
                hi = R - max(0, off)
                z = y[lo + off:hi + off, :]
                if lo or hi != R:
                    parts = []
                    if lo:
                        parts.append(jnp.zeros((lo, cout), f32))
                    parts.append(z)
                    if hi != R:
                        parts.append(jnp.zeros((R - hi, cout), f32))
                    z = jnp.concatenate(parts, axis=0)
                acc = z if acc is None else acc + z
        return jnp.maximum(acc + sh_ref[...], 0.0)

    t1 = stem[:, :DC2]
    t2 = stem[:, DC2:]
    b1 = conv3x3(t1, w1_ref, s1_ref, DC2)
    b2 = conv3x3(t2, w2_ref, s2_ref, DC)
    b2 = conv3x3(b2, w3_ref, s3_ref, DC)

    out = jnp.dot(b1.astype(bf16), wfa_ref[...], preferred_element_type=f32)
    out = out + jnp.dot(b2.astype(bf16), wfb_ref[...], preferred_element_type=f32)
    out = out + sf_ref[...]

    img_of_row = jax.lax.div(jax.lax.broadcasted_iota(jnp.int32, (B, R), 1), HW)
    sel_pool = jnp.where(img_of_row == jax.lax.broadcasted_iota(jnp.int32, (B, R), 0),
                         one, zero)
    pooled = jnp.dot(sel_pool, out, preferred_element_type=f32) * (1.0 / HW)
    se = jnp.maximum(jnp.dot(pooled, f1w_ref[...], preferred_element_type=f32)
                     + f1b_ref[...], 0.0)
    se = jax.nn.sigmoid(jnp.dot(se, f2w_ref[...], preferred_element_type=f32)
                        + f2b_ref[...])

    row_img = jax.lax.div(jax.lax.broadcasted_iota(jnp.int32, (R, B), 0), HW)
    sel_bcast = jnp.where(row_img == jax.lax.broadcasted_iota(jnp.int32, (R, B), 1),
                          one, zero)
    se_rows = jnp.dot(sel_bcast, se, preferred_element_type=f32)

    y = jnp.maximum(out * se_rows + x, 0.0)
    for b in range(B):
        o_ref[b] = jnp.transpose(y[b * HW:(b + 1) * HW, :], (1, 0))


def kernel(x, conv1_1_w, bn1_1_g, bn1_1_be, bn1_1_m, bn1_1_v, conv1_2_w,
           conv2_1_w, bn2_1_g, bn2_1_be, bn2_1_m, bn2_1_v, conv2_2_w,
           bn2_2_g, bn2_2_be, bn2_2_m, bn2_2_v, conv2_3_w,
           bn_concat_g, bn_concat_be, bn_concat_m, bn_concat_v,
           conv_w, bn_g, bn_be, bn_m, bn_v, fc1_w, fc1_b, fc2_w, fc2_b):
    f32, bf16 = jnp.float32, jnp.bfloat16
    N, Cin, H, W = x.shape
    HW = H * W
    DC2 = conv1_1_w.shape[1]
    DC = conv2_1_w.shape[1]
    DC3 = DC2 + DC
    Cout = conv_w.shape[1]
    Cr = fc1_w.shape[1]

    s11, h11 = _fold_bn(bn1_1_g, bn1_1_be, bn1_1_m, bn1_1_v)
    s21, h21 = _fold_bn(bn2_1_g, bn2_1_be, bn2_1_m, bn2_1_v)
    s22, h22 = _fold_bn(bn2_2_g, bn2_2_be, bn2_2_m, bn2_2_v)
    scat, hcat = _fold_bn(bn_concat_g, bn_concat_be, bn_concat_m, bn_concat_v)
    sbn, hbn = _fold_bn(bn_g, bn_be, bn_m, bn_v)

    ws = jnp.concatenate([conv1_1_w * s11, conv2_1_w * s21], axis=1).astype(bf16)
    bs = jnp.concatenate([h11, h21]).reshape(1, DC3)
    w1 = (_dense_grouped(conv1_2_w) * scat[:DC2]).astype(bf16)
    w2 = (_dense_grouped(conv2_2_w) * s22).astype(bf16)
    w3 = (_dense_grouped(conv2_3_w) * scat[DC2:]).astype(bf16)
    wfa = (conv_w[:DC2] * sbn).astype(bf16)
    wfb = (conv_w[DC2:] * sbn).astype(bf16)

    x2 = x.reshape(N, Cin, HW)
    B = next(b for b in (8, 6, 4, 3, 2, 1) if N % b == 0)
    R = B * HW
    G = N // B

    kfn = functools.partial(_block_kernel, H, W, B, DC2, DC)
    full = lambda i: (0, 0)
    full3 = lambda i: (0, 0, 0)
    flops = N * (2 * HW * Cin * DC3
                 + 2 * HW * 9 * (DC2 * DC2 + 2 * DC * DC)
                 + 2 * HW * DC3 * Cout
                 + 4 * Cout * Cr + 5 * HW * Cout)
    bytes_acc = (N * HW * (Cin + Cout) * 4
                 + 2 * (Cin * DC3 + 9 * (DC2 * DC2 + 2 * DC * DC) + DC3 * Cout)
                 + 4 * (DC3 + DC2 + 2 * DC + 2 * Cout + 2 * Cout * Cr + Cr))

    out = pl.pallas_call(
        kfn,
        out_shape=jax.ShapeDtypeStruct((N, Cout, HW), f32),
        grid=(G,),
        in_specs=[
            pl.BlockSpec((B, Cin, HW), lambda i: (i, 0, 0)),
            pl.BlockSpec((Cin, DC3), full),
            pl.BlockSpec((1, DC3), full),
            pl.BlockSpec((9, DC2, DC2), full3),
            pl.BlockSpec((1, DC2), full),
            pl.BlockSpec((9, DC, DC), full3),
            pl.BlockSpec((1, DC), full),
            pl.BlockSpec((9, DC, DC), full3),
            pl.BlockSpec((1, DC), full),
            pl.BlockSpec((DC2, Cout), full),
            pl.BlockSpec((DC, Cout), full),
            pl.BlockSpec((1, Cout), full),
            pl.BlockSpec((Cout, Cr), full),
            pl.BlockSpec((1, Cr), full),
            pl.BlockSpec((Cr, Cout), full),
            pl.BlockSpec((1, Cout), full),
        ],
        out_specs=pl.BlockSpec((B, Cout, HW), lambda i: (i, 0, 0)),
        compiler_params=pltpu.CompilerParams(
            dimension_semantics=("parallel",),
            vmem_limit_bytes=64 * 1024 * 1024),
        cost_estimate=pl.CostEstimate(flops=flops, transcendentals=N * Cout,
                                      bytes_accessed=bytes_acc),
    )(x2, ws, bs,
      w1, hcat[:DC2].reshape(1, DC2),
      w2, h22.reshape(1, DC),
      w3, hcat[DC2:].reshape(1, DC),
      wfa, wfb, hbn.reshape(1, Cout),
      fc1_w, fc1_b.reshape(1, Cr),
      fc2_w, fc2_b.reshape(1, Cout))

    return out.reshape(N, Cout, H, W)


# speedup vs baseline: 3.1324x; 1.4849x over previous
import functools

import jax
import jax.numpy as jnp
from jax.experimental import pallas as pl
from jax.experimental.pallas import tpu as pltpu


def _prep_kernel(DC2, DC, Cout, G1, G2,
                 c11_ref, c21_ref, w1c_ref, w2c_ref, w3c_ref, cw_ref,
                 g11_ref, be11_ref, m11_ref, v11_ref,
                 g21_ref, be21_ref, m21_ref, v21_ref,
                 g22_ref, be22_ref, m22_ref, v22_ref,
                 gct_ref, bect_ref, mct_ref, vct_ref,
                 gbn_ref, bebn_ref, mbn_ref, vbn_ref,
                 ws_ref, bs_ref, w1_ref, s1_ref, w2_ref, s2_ref,
                 w3_ref, s3_ref, wfa_ref, wfb_ref, sf_ref):
    f32 = jnp.float32
    bf16 = jnp.bfloat16

    def fold(g_ref, be_ref, m_ref, v_ref):
        sc = g_ref[...] * jax.lax.rsqrt(v_ref[...] + 1e-5)
        return sc, be_ref[...] - m_ref[...] * sc

    sc11, sh11 = fold(g11_ref, be11_ref, m11_ref, v11_ref)
    sc21, sh21 = fold(g21_ref, be21_ref, m21_ref, v21_ref)
    sc22, sh22 = fold(g22_ref, be22_ref, m22_ref, v22_ref)
    scct, shct = fold(gct_ref, bect_ref, mct_ref, vct_ref)
    scbn, shbn = fold(gbn_ref, bebn_ref, mbn_ref, vbn_ref)

    ws_ref[:, :DC2] = (c11_ref[...] * sc11).astype(bf16)
    ws_ref[:, DC2:] = (c21_ref[...] * sc21).astype(bf16)
    bs_ref[:, :DC2] = sh11
    bs_ref[:, DC2:] = sh21
    s1_ref[...] = shct[:, :DC2]
    s2_ref[...] = sh22
    s3_ref[...] = shct[:, DC2:]
    sf_ref[...] = shbn

    wfa_ref[...] = (cw_ref[:DC2, :] * scbn).astype(bf16)
    wfb_ref[...] = (cw_ref[DC2:, :] * scbn).astype(bf16)

    def expand(dst_ref, src_ref, n, ci, scale_row):
        groups = n // ci
        col = jax.lax.broadcasted_iota(jnp.int32, (n, n), 1)
        row = jax.lax.broadcasted_iota(jnp.int32, (n, n), 0)
        mask = jnp.where((row // ci) == (col // ci),
                         jnp.broadcast_to(scale_row, (n, n)), 0.0)
        tcol = jax.lax.broadcasted_iota(jnp.int32, (ci, n), 1)
        trow = jax.lax.broadcasted_iota(jnp.int32, (ci, n), 0)
        tpat = jnp.where(jax.lax.rem(tcol, ci) == trow, 1.0, 0.0).astype(bf16)
        for t in range(9):
            a = jnp.concatenate(
                [src_ref[(g * 9 + t) * ci:(g * 9 + t) * ci + ci, :]
                 for g in range(groups)], axis=0)
            d = jnp.dot(a.astype(bf16), tpat, preferred_element_type=f32)
            dst_ref[t] = (d * mask).astype(bf16)

    expand(w1_ref, w1c_ref, DC2, DC2 // G1, scct[:, :DC2])
    expand(w2_ref, w2c_ref, DC, DC // G2, sc22)
    expand(w3_ref, w3c_ref, DC, DC // G2, scct[:, DC2:])


def _rep_lanes(m, cout):
    k = cout // m.shape[1]
    if k == 1:
        return m
    return jnp.concatenate([m] * k, axis=1)


def _block_kernel(H, W, B, DC2, DC,
                  x_ref, ws_ref, bs_ref,
                  w1_ref, s1_ref, w2_ref, s2_ref, w3_ref, s3_ref,
                  wfa_ref, wfb_ref, sf_ref,
                  f1w_ref, f1b_ref, f2w_ref, f2b_ref,
                  o_ref):
    f32 = jnp.float32
    bf16 = jnp.bfloat16
    HW = H * W
    R = B * HW
    Cout = o_ref.shape[2]

    x = jnp.concatenate([jnp.transpose(x_ref[b], (1, 0)) for b in range(B)],
                        axis=0)
    stem = jnp.dot(x.astype(bf16), ws_ref[...], preferred_element_type=f32)
    stem = jnp.maximum(stem + bs_ref[...], 0.0)

    MW = min(128, DC)
    p = jax.lax.broadcasted_iota(jnp.int32, (R, MW), 0)
    w_pos = jax.lax.rem(p, W)
    h_pos = jax.lax.rem(jax.lax.div(p, W), H)
    one, zero = jnp.float32(1.0), jnp.float32(0.0)
    m_h = [jnp.where(h_pos <= H - 2, one, zero), None,
           jnp.where(h_pos >= 1, one, zero)]
    m_w = [jnp.where(w_pos <= W - 2, one, zero), None,
           jnp.where(w_pos >= 1, one, zero)]

    def conv3x3(src, w9_ref, sh_ref, cout):
        srcb = src.astype(bf16)
        acc = None
        for dy in range(3):
            for dx in range(3):
                off = (dy - 1) * W + (dx - 1)
                y = jnp.dot(srcb, w9_ref[dy * 3 + dx],
                            preferred_element_type=f32)
                if m_h[dy] is not None:
                    y = y * _rep_lanes(m_h[dy], cout)
                if m_w[dx] is not None:
                    y = y * _rep_lanes(m_w[dx], cout)
                lo = max(0, -off)
                hi = R - max(0, off)
                z = y[lo + off:hi + off, :]
                if lo or hi != R:
                    parts = []
                    if lo:
                        parts.append(jnp.zeros((lo, cout), f32))
                    parts.append(z)
                    if hi != R:
                        parts.append(jnp.zeros((R - hi, cout), f32))
                    z = jnp.concatenate(parts, axis=0)
                acc = z if acc is None else acc + z
        return jnp.maximum(acc + sh_ref[...], 0.0)

    t1 = stem[:, :DC2]
    t2 = stem[:, DC2:]
    b1 = conv3x3(t1, w1_ref, s1_ref, DC2)
    b2 = conv3x3(t2, w2_ref, s2_ref, DC)
    b2 = conv3x3(b2, w3_ref, s3_ref, DC)

    out = jnp.dot(b1.astype(bf16), wfa_ref[...], preferred_element_type=f32)
    out = out + jnp.dot(b2.astype(bf16), wfb_ref[...], preferred_element_type=f32)
    out = out + sf_ref[...]

    img_of_row = jax.lax.div(jax.lax.broadcasted_iota(jnp.int32, (B, R), 1), HW)
    sel_pool = jnp.where(img_of_row == jax.lax.broadcasted_iota(jnp.int32, (B, R), 0),
                         one, zero)
    pooled = jnp.dot(sel_pool, out, preferred_element_type=f32) * (1.0 / HW)
    se = jnp.maximum(jnp.dot(pooled, f1w_ref[...], preferred_element_type=f32)
                     + f1b_ref[...], 0.0)
    se = jax.nn.sigmoid(jnp.dot(se, f2w_ref[...], preferred_element_type=f32)
                        + f2b_ref[...])

    row_img = jax.lax.div(jax.lax.broadcasted_iota(jnp.int32, (R, B), 0), HW)
    sel_bcast = jnp.where(row_img == jax.lax.broadcasted_iota(jnp.int32, (R, B), 1),
                          one, zero)
    se_rows = jnp.dot(sel_bcast, se, preferred_element_type=f32)

    y = jnp.maximum(out * se_rows + x, 0.0)
    for b in range(B):
        o_ref[b] = jnp.transpose(y[b * HW:(b + 1) * HW, :], (1, 0))


def kernel(x, conv1_1_w, bn1_1_g, bn1_1_be, bn1_1_m, bn1_1_v, conv1_2_w,
           conv2_1_w, bn2_1_g, bn2_1_be, bn2_1_m, bn2_1_v, conv2_2_w,
           bn2_2_g, bn2_2_be, bn2_2_m, bn2_2_v, conv2_3_w,
           bn_concat_g, bn_concat_be, bn_concat_m, bn_concat_v,
           conv_w, bn_g, bn_be, bn_m, bn_v, fc1_w, fc1_b, fc2_w, fc2_b):
    f32, bf16 = jnp.float32, jnp.bfloat16
    N, Cin, H, W = x.shape
    HW = H * W
    DC2 = conv1_1_w.shape[1]
    DC = conv2_1_w.shape[1]
    DC3 = DC2 + DC
    Cout = conv_w.shape[1]
    Cr = fc1_w.shape[1]

    G1 = conv1_2_w.shape[0]
    G2 = conv2_2_w.shape[0]
    ci1 = DC2 // G1
    ci2 = DC // G2

    prep = functools.partial(_prep_kernel, DC2, DC, Cout, G1, G2)
    row = lambda a: a.reshape(1, -1)
    full2 = lambda i: (0, 0)
    full3 = lambda i: (0, 0, 0)
    vec_in = [pl.BlockSpec((1, s.shape[0]), full2) for s in
              (bn1_1_g, bn1_1_be, bn1_1_m, bn1_1_v,
               bn2_1_g, bn2_1_be, bn2_1_m, bn2_1_v,
               bn2_2_g, bn2_2_be, bn2_2_m, bn2_2_v,
               bn_concat_g, bn_concat_be, bn_concat_m, bn_concat_v,
               bn_g, bn_be, bn_m, bn_v)]
    ws, bs, w1, s1, w2, s2, w3, s3, wfa, wfb, sf = pl.pallas_call(
        prep,
        out_shape=(
            jax.ShapeDtypeStruct((Cin, DC3), bf16),
            jax.ShapeDtypeStruct((1, DC3), f32),
            jax.ShapeDtypeStruct((9, DC2, DC2), bf16),
            jax.ShapeDtypeStruct((1, DC2), f32),
            jax.ShapeDtypeStruct((9, DC, DC), bf16),
            jax.ShapeDtypeStruct((1, DC), f32),
            jax.ShapeDtypeStruct((9, DC, DC), bf16),
            jax.ShapeDtypeStruct((1, DC), f32),
            jax.ShapeDtypeStruct((DC2, Cout), bf16),
            jax.ShapeDtypeStruct((DC, Cout), bf16),
            jax.ShapeDtypeStruct((1, Cout), f32),
        ),
        grid=(1,),
        in_specs=[
            pl.BlockSpec((Cin, DC2), full2),
            pl.BlockSpec((Cin, DC), full2),
            pl.BlockSpec((G1 * 9 * ci1, ci1), full2),
            pl.BlockSpec((G2 * 9 * ci2, ci2), full2),
            pl.BlockSpec((G2 * 9 * ci2, ci2), full2),
            pl.BlockSpec((DC3, Cout), full2),
        ] + vec_in,
        out_specs=(
            pl.BlockSpec((Cin, DC3), full2),
            pl.BlockSpec((1, DC3), full2),
            pl.BlockSpec((9, DC2, DC2), full3),
            pl.BlockSpec((1, DC2), full2),
            pl.BlockSpec((9, DC, DC), full3),
            pl.BlockSpec((1, DC), full2),
            pl.BlockSpec((9, DC, DC), full3),
            pl.BlockSpec((1, DC), full2),
            pl.BlockSpec((DC2, Cout), full2),
            pl.BlockSpec((DC, Cout), full2),
            pl.BlockSpec((1, Cout), full2),
        ),
        compiler_params=pltpu.CompilerParams(
            dimension_semantics=("arbitrary",),
            vmem_limit_bytes=64 * 1024 * 1024),
    )(conv1_1_w, conv2_1_w,
      conv1_2_w.reshape(G1 * 9 * ci1, ci1),
      conv2_2_w.reshape(G2 * 9 * ci2, ci2),
      conv2_3_w.reshape(G2 * 9 * ci2, ci2),
      conv_w,
      row(bn1_1_g), row(bn1_1_be), row(bn1_1_m), row(bn1_1_v),
      row(bn2_1_g), row(bn2_1_be), row(bn2_1_m), row(bn2_1_v),
      row(bn2_2_g), row(bn2_2_be), row(bn2_2_m), row(bn2_2_v),
      row(bn_concat_g), row(bn_concat_be), row(bn_concat_m), row(bn_concat_v),
      row(bn_g), row(bn_be), row(bn_m), row(bn_v))

    x2 = x.reshape(N, Cin, HW)
    B = next(b for b in (8, 6, 4, 3, 2, 1) if N % b == 0)
    R = B * HW
    G = N // B

    kfn = functools.partial(_block_kernel, H, W, B, DC2, DC)
    full = lambda i: (0, 0)
    full3 = lambda i: (0, 0, 0)
    flops = N * (2 * HW * Cin * DC3
                 + 2 * HW * 9 * (DC2 * DC2 + 2 * DC * DC)
                 + 2 * HW * DC3 * Cout
                 + 4 * Cout * Cr + 5 * HW * Cout)
    bytes_acc = (N * HW * (Cin + Cout) * 4
                 + 2 * (Cin * DC3 + 9 * (DC2 * DC2 + 2 * DC * DC) + DC3 * Cout)
                 + 4 * (DC3 + DC2 + 2 * DC + 2 * Cout + 2 * Cout * Cr + Cr))

    out = pl.pallas_call(
        kfn,
        out_shape=jax.ShapeDtypeStruct((N, Cout, HW), f32),
        grid=(G,),
        in_specs=[
            pl.BlockSpec((B, Cin, HW), lambda i: (i, 0, 0)),
            pl.BlockSpec((Cin, DC3), full),
            pl.BlockSpec((1, DC3), full),
            pl.BlockSpec((9, DC2, DC2), full3),
            pl.BlockSpec((1, DC2), full),
            pl.BlockSpec((9, DC, DC), full3),
            pl.BlockSpec((1, DC), full),
            pl.BlockSpec((9, DC, DC), full3),
            pl.BlockSpec((1, DC), full),
            pl.BlockSpec((DC2, Cout), full),
            pl.BlockSpec((DC, Cout), full),
            pl.BlockSpec((1, Cout), full),
            pl.BlockSpec((Cout, Cr), full),
            pl.BlockSpec((1, Cr), full),
            pl.BlockSpec((Cr, Cout), full),
            pl.BlockSpec((1, Cout), full),
        ],
        out_specs=pl.BlockSpec((B, Cout, HW), lambda i: (i, 0, 0)),
        compiler_params=pltpu.CompilerParams(
            dimension_semantics=("parallel",),
            vmem_limit_bytes=64 * 1024 * 1024),
        cost_estimate=pl.CostEstimate(flops=flops, transcendentals=N * Cout,
                                      bytes_accessed=bytes_acc),
    )(x2, ws, bs,
      w1, s1,
      w2, s2,
      w3, s3,
      wfa, wfb, sf,
      fc1_w, fc1_b.reshape(1, Cr),
      fc2_w, fc2_b.reshape(1, Cout))

    return out.reshape(N, Cout, H, W)


# speedup vs baseline: 3.2101x; 1.0248x over previous
import functools

import jax
import jax.numpy as jnp
from jax.experimental import pallas as pl
from jax.experimental.pallas import tpu as pltpu


def _rep_lanes(m, cout):
    k = cout // m.shape[1]
    if k == 1:
        return m
    return jnp.concatenate([m] * k, axis=1)


def _block_kernel(H, W, B, DC2, DC, G1, G2,
                  x_ref,
                  c11_ref, c21_ref, w1c_ref, w2c_ref, w3c_ref, cw_ref,
                  g11_ref, be11_ref, m11_ref, v11_ref,
                  g21_ref, be21_ref, m21_ref, v21_ref,
                  g22_ref, be22_ref, m22_ref, v22_ref,
                  gct_ref, bect_ref, mct_ref, vct_ref,
                  gbn_ref, bebn_ref, mbn_ref, vbn_ref,
                  f1w_ref, f1b_ref, f2w_ref, f2b_ref,
                  o_ref):
    f32 = jnp.float32
    bf16 = jnp.bfloat16
    HW = H * W
    R = B * HW
    Cout = o_ref.shape[1]

    def fold(g_ref, be_ref, m_ref, v_ref):
        sc = g_ref[...] * jax.lax.rsqrt(v_ref[...] + 1e-5)
        return sc, be_ref[...] - m_ref[...] * sc

    sc11, sh11 = fold(g11_ref, be11_ref, m11_ref, v11_ref)
    sc21, sh21 = fold(g21_ref, be21_ref, m21_ref, v21_ref)
    sc22, s2 = fold(g22_ref, be22_ref, m22_ref, v22_ref)
    scct, shct = fold(gct_ref, bect_ref, mct_ref, vct_ref)
    scbn, sf = fold(gbn_ref, bebn_ref, mbn_ref, vbn_ref)

    ws = jnp.concatenate([(c11_ref[...] * sc11).astype(bf16),
                          (c21_ref[...] * sc21).astype(bf16)], axis=1)
    bs = jnp.concatenate([sh11, sh21], axis=1)
    s1 = shct[:, :DC2]
    s3 = shct[:, DC2:]
    wfa = (cw_ref[:DC2, :] * scbn).astype(bf16)
    wfb = (cw_ref[DC2:, :] * scbn).astype(bf16)

    def expand(src_ref, n, ci, scale_row):
        groups = n // ci
        col = jax.lax.broadcasted_iota(jnp.int32, (n, n), 1)
        row = jax.lax.broadcasted_iota(jnp.int32, (n, n), 0)
        mask = jnp.where((row // ci) == (col // ci),
                         jnp.broadcast_to(scale_row, (n, n)), 0.0)
        tcol = jax.lax.broadcasted_iota(jnp.int32, (ci, n), 1)
        trow = jax.lax.broadcasted_iota(jnp.int32, (ci, n), 0)
        tpat = jnp.where(jax.lax.rem(tcol, ci) == trow, 1.0, 0.0).astype(bf16)
        dense = []
        for t in range(9):
            a = jnp.concatenate(
                [src_ref[(g * 9 + t) * ci:(g * 9 + t) * ci + ci, :]
                 for g in range(groups)], axis=0)
            d = jnp.dot(a.astype(bf16), tpat, preferred_element_type=f32)
            dense.append((d * mask).astype(bf16))
        return dense

    w1 = expand(w1c_ref, DC2, DC2 // G1, scct[:, :DC2])
    w2 = expand(w2c_ref, DC, DC // G2, sc22)
    w3 = expand(w3c_ref, DC, DC // G2, scct[:, DC2:])

    x = jnp.concatenate([jnp.transpose(x_ref[b], (1, 0)) for b in range(B)],
                        axis=0)
    stem = jnp.dot(x.astype(bf16), ws, preferred_element_type=f32)
    stem = jnp.maximum(stem + bs, 0.0)

    MW = min(128, DC)
    p = jax.lax.broadcasted_iota(jnp.int32, (R, MW), 0)
    w_pos = jax.lax.rem(p, W)
    h_pos = jax.lax.rem(jax.lax.div(p, W), H)
    one, zero = jnp.float32(1.0), jnp.float32(0.0)
    m_h = [jnp.where(h_pos <= H - 2, one, zero), None,
           jnp.where(h_pos >= 1, one, zero)]
    m_w = [jnp.where(w_pos <= W - 2, one, zero), None,
           jnp.where(w_pos >= 1, one, zero)]

    def conv3x3(src, w9, sh, cout):
        srcb = src.astype(bf16)
        acc = None
        for dy in range(3):
            for dx in range(3):
                off = (dy - 1) * W + (dx - 1)
                y = jnp.dot(srcb, w9[dy * 3 + dx],
                            preferred_element_type=f32)
                if m_h[dy] is not None:
                    y = y * _rep_lanes(m_h[dy], cout)
                if m_w[dx] is not None:
                    y = y * _rep_lanes(m_w[dx], cout)
                lo = max(0, -off)
                hi = R - max(0, off)
                z = y[lo + off:hi + off, :]
                if lo or hi != R:
                    parts = []
                    if lo:
                        parts.append(jnp.zeros((lo, cout), f32))
                    parts.append(z)
                    if hi != R:
                        parts.append(jnp.zeros((R - hi, cout), f32))
                    z = jnp.concatenate(parts, axis=0)
                acc = z if acc is None else acc + z
        return jnp.maximum(acc + sh, 0.0)

    t1 = stem[:, :DC2]
    t2 = stem[:, DC2:]
    b1 = conv3x3(t1, w1, s1, DC2)
    b2 = conv3x3(t2, w2, s2, DC)
    b2 = conv3x3(b2, w3, s3, DC)

    out = jnp.dot(b1.astype(bf16), wfa, preferred_element_type=f32)
    out = out + jnp.dot(b2.astype(bf16), wfb, preferred_element_type=f32)
    out = out + sf

    img_of_row = jax.lax.div(jax.lax.broadcasted_iota(jnp.int32, (B, R), 1), HW)
    sel_pool = jnp.where(img_of_row == jax.lax.broadcasted_iota(jnp.int32, (B, R), 0),
                         one, zero)
    pooled = jnp.dot(sel_pool, out, preferred_element_type=f32) * (1.0 / HW)
    se = jnp.maximum(jnp.dot(pooled, f1w_ref[...], preferred_element_type=f32)
                     + f1b_ref[...], 0.0)
    se = jax.nn.sigmoid(jnp.dot(se, f2w_ref[...], preferred_element_type=f32)
                        + f2b_ref[...])

    row_img = jax.lax.div(jax.lax.broadcasted_iota(jnp.int32, (R, B), 0), HW)
    sel_bcast = jnp.where(row_img == jax.lax.broadcasted_iota(jnp.int32, (R, B), 1),
                          one, zero)
    se_rows = jnp.dot(sel_bcast, se, preferred_element_type=f32)

    y = jnp.maximum(out * se_rows + x, 0.0)
    for b in range(B):
        o_ref[b] = jnp.transpose(y[b * HW:(b + 1) * HW, :], (1, 0))


def kernel(x, conv1_1_w, bn1_1_g, bn1_1_be, bn1_1_m, bn1_1_v, conv1_2_w,
           conv2_1_w, bn2_1_g, bn2_1_be, bn2_1_m, bn2_1_v, conv2_2_w,
           bn2_2_g, bn2_2_be, bn2_2_m, bn2_2_v, conv2_3_w,
           bn_concat_g, bn_concat_be, bn_concat_m, bn_concat_v,
           conv_w, bn_g, bn_be, bn_m, bn_v, fc1_w, fc1_b, fc2_w, fc2_b):
    f32 = jnp.float32
    N, Cin, H, W = x.shape
    HW = H * W
    DC2 = conv1_1_w.shape[1]
    DC = conv2_1_w.shape[1]
    DC3 = DC2 + DC
    Cout = conv_w.shape[1]
    Cr = fc1_w.shape[1]
    G1 = conv1_2_w.shape[0]
    G2 = conv2_2_w.shape[0]
    ci1 = DC2 // G1
    ci2 = DC // G2

    x2 = x.reshape(N, Cin, HW)
    B = next(b for b in (8, 6, 4, 3, 2, 1) if N % b == 0)
    G = N // B

    kfn = functools.partial(_block_kernel, H, W, B, DC2, DC, G1, G2)
    row = lambda a: a.reshape(1, -1)
    full = lambda i: (0, 0)
    vspec = lambda c: pl.BlockSpec((1, c), full)
    flops = N * (2 * HW * Cin * DC3
                 + 2 * HW * 9 * (DC2 * DC2 + 2 * DC * DC)
                 + 2 * HW * DC3 * Cout
                 + 4 * Cout * Cr + 5 * HW * Cout)
    bytes_acc = (N * HW * (Cin + Cout) * 4
                 + 4 * (Cin * DC3 + 9 * (G1 * ci1 * ci1 + 2 * G2 * ci2 * ci2)
                        + DC3 * Cout + 2 * Cout * Cr))

    out = pl.pallas_call(
        kfn,
        out_shape=jax.ShapeDtypeStruct((N, Cout, HW), f32),
        grid=(G,),
        in_specs=[
            pl.BlockSpec((B, Cin, HW), lambda i: (i, 0, 0)),
            pl.BlockSpec((Cin, DC2), full),
            pl.BlockSpec((Cin, DC), full),
            pl.BlockSpec((G1 * 9 * ci1, ci1), full),
            pl.BlockSpec((G2 * 9 * ci2, ci2), full),
            pl.BlockSpec((G2 * 9 * ci2, ci2), full),
            pl.BlockSpec((DC3, Cout), full),
        ] + [vspec(DC2)] * 4 + [vspec(DC)] * 8 + [vspec(DC3)] * 4
          + [vspec(Cout)] * 4 + [
            pl.BlockSpec((Cout, Cr), full),
            vspec(Cr),
            pl.BlockSpec((Cr, Cout), full),
            vspec(Cout),
        ],
        out_specs=pl.BlockSpec((B, Cout, HW), lambda i: (i, 0, 0)),
        compiler_params=pltpu.CompilerParams(
            dimension_semantics=("parallel",),
            vmem_limit_bytes=64 * 1024 * 1024),
        cost_estimate=pl.CostEstimate(flops=flops, transcendentals=N * Cout,
                                      bytes_accessed=bytes_acc),
    )(x2,
      conv1_1_w, conv2_1_w,
      conv1_2_w.reshape(G1 * 9 * ci1, ci1),
      conv2_2_w.reshape(G2 * 9 * ci2, ci2),
      conv2_3_w.reshape(G2 * 9 * ci2, ci2),
      conv_w,
      row(bn1_1_g), row(bn1_1_be), row(bn1_1_m), row(bn1_1_v),
      row(bn2_1_g), row(bn2_1_be), row(bn2_1_m), row(bn2_1_v),
      row(bn2_2_g), row(bn2_2_be), row(bn2_2_m), row(bn2_2_v),
      row(bn_concat_g), row(bn_concat_be), row(bn_concat_m), row(bn_concat_v),
      row(bn_g), row(bn_be), row(bn_m), row(bn_v),
      fc1_w, row(fc1_b), fc2_w, row(fc2_b))

    return out.reshape(N, Cout, H, W)
